# baseline (device time: 1321261 ns/iter reference)
import functools

import jax
import jax.numpy as jnp
from jax import lax
from jax.experimental import pallas as pl
from jax.experimental.pallas import tpu as pltpu

N_DEV = 8
N_HOPS = N_DEV - 1
MESH = pltpu.DeviceIdType.MESH


def _fused_agattn(Qr, KA, KB, VA, VB):
    BH, S, D = Qr.shape
    H2 = S // 2
    scale = D ** -0.5
    f32 = jnp.float32

    def flash_unit(i, q, k_s, v_s, m_ref, l_ref, out_ref):
        k = k_s[i]
        v = v_s[i]
        s_qk = lax.dot_general(
            q, k, (((1,), (1,)), ((), ())), preferred_element_type=f32
        ) * scale
        m_prev = m_ref[i]
        m_new = jnp.maximum(m_prev, jnp.max(s_qk, axis=-1))
        alpha = jnp.exp(m_prev - m_new)
        p = jnp.exp(s_qk - m_new[:, None])
        l_ref[i] = l_ref[i] * alpha + jnp.sum(p, axis=-1)
        pv = lax.dot_general(
            p, v, (((1,), (0,)), ((), ())), preferred_element_type=f32
        )
        out_ref[i] = out_ref[i] * alpha[:, None] + pv
        m_ref[i] = m_new

    def body(q_ref, ka_ref, kb_ref, va_ref, vb_ref,
             out_ref, kgA, kgB, vgA, vgB,
             kA_s, kB_s, vA_s, vB_s, m_ref, l_ref,
             copy_sems, stage_sems,
             sA_k, rA_k, sA_v, rA_v, sB_k, rB_k, sB_v, rB_v):
        my = lax.axis_index("i")
        left = lax.rem(my - 1 + N_DEV, N_DEV)
        right = lax.rem(my + 1, N_DEV)

        barrier_sem = pltpu.get_barrier_semaphore()
        for nbr in (left, right):
            pl.semaphore_signal(barrier_sem, inc=1, device_id=(nbr,),
                                device_id_type=MESH)
        pl.semaphore_wait(barrier_sem, 2)

        own = [
            pltpu.make_async_copy(ka_ref, kgA.at[my], copy_sems.at[0]),
            pltpu.make_async_copy(kb_ref, kgB.at[my], copy_sems.at[1]),
            pltpu.make_async_copy(va_ref, vgA.at[my], copy_sems.at[2]),
            pltpu.make_async_copy(vb_ref, vgB.at[my], copy_sems.at[3]),
        ]
        stage = [
            pltpu.make_async_copy(ka_ref, kA_s, stage_sems.at[0]),
            pltpu.make_async_copy(kb_ref, kB_s, stage_sems.at[1]),
            pltpu.make_async_copy(va_ref, vA_s, stage_sems.at[2]),
            pltpu.make_async_copy(vb_ref, vB_s, stage_sems.at[3]),
        ]
        for c in own + stage:
            c.start()

        m_ref[...] = jnp.full((BH, S), -jnp.inf, f32)
        l_ref[...] = jnp.zeros((BH, S), f32)
        out_ref[...] = jnp.zeros((BH, S, D), f32)

        for c in own + stage:
            c.wait()

        for t in range(N_HOPS):
            cR = lax.rem(my - t + N_DEV, N_DEV)
            cL = lax.rem(my + t + N_DEV, N_DEV)
            rdmas = [
                pltpu.make_async_remote_copy(
                    src_ref=kgA.at[cR], dst_ref=kgA.at[cR],
                    send_sem=sA_k.at[t], recv_sem=rA_k.at[t],
                    device_id=(right,), device_id_type=MESH),
                pltpu.make_async_remote_copy(
                    src_ref=vgA.at[cR], dst_ref=vgA.at[cR],
                    send_sem=sA_v.at[t], recv_sem=rA_v.at[t],
                    device_id=(right,), device_id_type=MESH),
                pltpu.make_async_remote_copy(
                    src_ref=kgB.at[cL], dst_ref=kgB.at[cL],
                    send_sem=sB_k.at[t], recv_sem=rB_k.at[t],
                    device_id=(left,), device_id_type=MESH),
                pltpu.make_async_remote_copy(
                    src_ref=vgB.at[cL], dst_ref=vgB.at[cL],
                    send_sem=sB_v.at[t], recv_sem=rB_v.at[t],
                    device_id=(left,), device_id_type=MESH),
            ]
            for r in rdmas:
                r.start()

            def compute(i, _):
                q = q_ref[i]
                flash_unit(i, q, kA_s, vA_s, m_ref, l_ref, out_ref)
                flash_unit(i, q, kB_s, vB_s, m_ref, l_ref, out_ref)
                return 0
            lax.fori_loop(0, BH, compute, 0)

            for r in rdmas:
                r.wait()

            aR = lax.rem(my - t - 1 + N_DEV, N_DEV)
            aL = lax.rem(my + t + 1 + N_DEV, N_DEV)
            stage = [
                pltpu.make_async_copy(kgA.at[aR], kA_s, stage_sems.at[0]),
                pltpu.make_async_copy(vgA.at[aR], vA_s, stage_sems.at[1]),
                pltpu.make_async_copy(kgB.at[aL], kB_s, stage_sems.at[2]),
                pltpu.make_async_copy(vgB.at[aL], vB_s, stage_sems.at[3]),
            ]
            for c in stage:
                c.start()
            for c in stage:
                c.wait()

        def compute_tail(i, _):
            q = q_ref[i]
            flash_unit(i, q, kA_s, vA_s, m_ref, l_ref, out_ref)
            flash_unit(i, q, kB_s, vB_s, m_ref, l_ref, out_ref)
            return 0
        lax.fori_loop(0, BH, compute_tail, 0)

        out_ref[...] = out_ref[...] / l_ref[...][:, :, None]

        @functools.partial(pl.run_scoped,
                           exit_sem=pltpu.SemaphoreType.REGULAR)
        def _(exit_sem):
            for nbr in (left, right):
                pl.semaphore_signal(exit_sem, inc=1, device_id=(nbr,),
                                    device_id_type=MESH)
            pl.semaphore_wait(exit_sem, 2)

    hbm = pl.BlockSpec(memory_space=pltpu.MemorySpace.HBM)
    vmem = pl.BlockSpec(memory_space=pltpu.MemorySpace.VMEM)
    half = jax.ShapeDtypeStruct((N_DEV, BH, H2, D), f32)
    out, _, _, _, _ = pl.pallas_call(
        body,
        out_shape=(
            jax.ShapeDtypeStruct((BH, S, D), f32),
            half, half, half, half,
        ),
        in_specs=[vmem, hbm, hbm, hbm, hbm],
        out_specs=(vmem, hbm, hbm, hbm, hbm),
        scratch_shapes=[
            pltpu.VMEM((BH, H2, D), f32),
            pltpu.VMEM((BH, H2, D), f32),
            pltpu.VMEM((BH, H2, D), f32),
            pltpu.VMEM((BH, H2, D), f32),
            pltpu.VMEM((BH, S), f32),
            pltpu.VMEM((BH, S), f32),
            pltpu.SemaphoreType.DMA((4,)),
            pltpu.SemaphoreType.DMA((4,)),
            pltpu.SemaphoreType.DMA((N_HOPS,)),
            pltpu.SemaphoreType.DMA((N_HOPS,)),
            pltpu.SemaphoreType.DMA((N_HOPS,)),
            pltpu.SemaphoreType.DMA((N_HOPS,)),
            pltpu.SemaphoreType.DMA((N_HOPS,)),
            pltpu.SemaphoreType.DMA((N_HOPS,)),
            pltpu.SemaphoreType.DMA((N_HOPS,)),
            pltpu.SemaphoreType.DMA((N_HOPS,)),
        ],
        compiler_params=pltpu.CompilerParams(
            collective_id=0, vmem_limit_bytes=56 * 1024 * 1024
        ),
    )(Qr, KA, KB, VA, VB)
    return out


def kernel(Q, K, V):
    b, s, h, d = Q.shape
    BH = b * h
    H2 = s // 2

    def to_bhsd(x):
        return x.transpose(0, 2, 1, 3).reshape(BH, x.shape[1], d)

    Qr = to_bhsd(Q)
    KA = to_bhsd(K[:, :H2])
    KB = to_bhsd(K[:, H2:])
    VA = to_bhsd(V[:, :H2])
    VB = to_bhsd(V[:, H2:])

    out = _fused_agattn(Qr, KA, KB, VA, VB)
    return out.reshape(b, h, s, d).transpose(0, 2, 1, 3).astype(Q.dtype)


# device time: 1247376 ns/iter; 1.0592x vs baseline; 1.0592x over previous
import functools

import jax
import jax.numpy as jnp
from jax import lax
from jax.experimental import pallas as pl
from jax.experimental.pallas import tpu as pltpu

N_DEV = 8
N_HOPS = N_DEV - 1
MESH = pltpu.DeviceIdType.MESH


def _fused_agattn(Qr, KA, KB, VA, VB):
    BH, S, D = Qr.shape
    H2 = S // 2
    scale = D ** -0.5
    f32 = jnp.float32

    NSPLIT = 2
    G = BH // NSPLIT

    def accum_unit(q_ref, k_s, v_s, l_ref, out_ref):
        for g in range(NSPLIT):
            sl = pl.ds(g * G, G)
            q = q_ref[sl]
            k = k_s[sl]
            v = v_s[sl]
            s_qk = lax.dot_general(
                q, k, (((2,), (2,)), ((0,), (0,))),
                preferred_element_type=f32,
            ) * scale
            p = jnp.exp(s_qk)
            l_ref[sl] = l_ref[sl] + jnp.sum(p, axis=-1)
            pv = lax.dot_general(
                p, v, (((2,), (1,)), ((0,), (0,))),
                preferred_element_type=f32,
            )
            out_ref[sl] = out_ref[sl] + pv

    def body(q_ref, ka_ref, kb_ref, va_ref, vb_ref,
             out_ref, kgA, kgB, vgA, vgB,
             kA_s, kB_s, vA_s, vB_s, l_ref,
             copy_sems, stage_sems,
             sA_k, rA_k, sA_v, rA_v, sB_k, rB_k, sB_v, rB_v):
        my = lax.axis_index("i")
        left = lax.rem(my - 1 + N_DEV, N_DEV)
        right = lax.rem(my + 1, N_DEV)

        barrier_sem = pltpu.get_barrier_semaphore()
        for nbr in (left, right):
            pl.semaphore_signal(barrier_sem, inc=1, device_id=(nbr,),
                                device_id_type=MESH)
        pl.semaphore_wait(barrier_sem, 2)

        own = [
            pltpu.make_async_copy(ka_ref, kgA.at[my], copy_sems.at[0]),
            pltpu.make_async_copy(kb_ref, kgB.at[my], copy_sems.at[1]),
            pltpu.make_async_copy(va_ref, vgA.at[my], copy_sems.at[2]),
            pltpu.make_async_copy(vb_ref, vgB.at[my], copy_sems.at[3]),
        ]
        stage = [
            pltpu.make_async_copy(ka_ref, kA_s, stage_sems.at[0]),
            pltpu.make_async_copy(kb_ref, kB_s, stage_sems.at[1]),
            pltpu.make_async_copy(va_ref, vA_s, stage_sems.at[2]),
            pltpu.make_async_copy(vb_ref, vB_s, stage_sems.at[3]),
        ]
        for c in own + stage:
            c.start()

        l_ref[...] = jnp.zeros((BH, S), f32)
        out_ref[...] = jnp.zeros((BH, S, D), f32)

        for c in own + stage:
            c.wait()

        for t in range(N_HOPS):
            cR = lax.rem(my - t + N_DEV, N_DEV)
            cL = lax.rem(my + t + N_DEV, N_DEV)
            rdmas = [
                pltpu.make_async_remote_copy(
                    src_ref=kgA.at[cR], dst_ref=kgA.at[cR],
                    send_sem=sA_k.at[t], recv_sem=rA_k.at[t],
                    device_id=(right,), device_id_type=MESH),
                pltpu.make_async_remote_copy(
                    src_ref=vgA.at[cR], dst_ref=vgA.at[cR],
                    send_sem=sA_v.at[t], recv_sem=rA_v.at[t],
                    device_id=(right,), device_id_type=MESH),
                pltpu.make_async_remote_copy(
                    src_ref=kgB.at[cL], dst_ref=kgB.at[cL],
                    send_sem=sB_k.at[t], recv_sem=rB_k.at[t],
                    device_id=(left,), device_id_type=MESH),
                pltpu.make_async_remote_copy(
                    src_ref=vgB.at[cL], dst_ref=vgB.at[cL],
                    send_sem=sB_v.at[t], recv_sem=rB_v.at[t],
                    device_id=(left,), device_id_type=MESH),
            ]
            for r in rdmas:
                r.start()

            accum_unit(q_ref, kA_s, vA_s, l_ref, out_ref)
            accum_unit(q_ref, kB_s, vB_s, l_ref, out_ref)

            for r in rdmas:
                r.wait()

            aR = lax.rem(my - t - 1 + N_DEV, N_DEV)
            aL = lax.rem(my + t + 1 + N_DEV, N_DEV)
            stage = [
                pltpu.make_async_copy(kgA.at[aR], kA_s, stage_sems.at[0]),
                pltpu.make_async_copy(vgA.at[aR], vA_s, stage_sems.at[1]),
                pltpu.make_async_copy(kgB.at[aL], kB_s, stage_sems.at[2]),
                pltpu.make_async_copy(vgB.at[aL], vB_s, stage_sems.at[3]),
            ]
            for c in stage:
                c.start()
            for c in stage:
                c.wait()

        accum_unit(q_ref, kA_s, vA_s, l_ref, out_ref)
        accum_unit(q_ref, kB_s, vB_s, l_ref, out_ref)

        out_ref[...] = out_ref[...] / l_ref[...][:, :, None]

        @functools.partial(pl.run_scoped,
                           exit_sem=pltpu.SemaphoreType.REGULAR)
        def _(exit_sem):
            for nbr in (left, right):
                pl.semaphore_signal(exit_sem, inc=1, device_id=(nbr,),
                                    device_id_type=MESH)
            pl.semaphore_wait(exit_sem, 2)

    hbm = pl.BlockSpec(memory_space=pltpu.MemorySpace.HBM)
    vmem = pl.BlockSpec(memory_space=pltpu.MemorySpace.VMEM)
    half = jax.ShapeDtypeStruct((N_DEV, BH, H2, D), f32)
    out, _, _, _, _ = pl.pallas_call(
        body,
        out_shape=(
            jax.ShapeDtypeStruct((BH, S, D), f32),
            half, half, half, half,
        ),
        in_specs=[vmem, hbm, hbm, hbm, hbm],
        out_specs=(vmem, hbm, hbm, hbm, hbm),
        scratch_shapes=[
            pltpu.VMEM((BH, H2, D), f32),
            pltpu.VMEM((BH, H2, D), f32),
            pltpu.VMEM((BH, H2, D), f32),
            pltpu.VMEM((BH, H2, D), f32),
            pltpu.VMEM((BH, S), f32),
            pltpu.SemaphoreType.DMA((4,)),
            pltpu.SemaphoreType.DMA((4,)),
            pltpu.SemaphoreType.DMA((N_HOPS,)),
            pltpu.SemaphoreType.DMA((N_HOPS,)),
            pltpu.SemaphoreType.DMA((N_HOPS,)),
            pltpu.SemaphoreType.DMA((N_HOPS,)),
            pltpu.SemaphoreType.DMA((N_HOPS,)),
            pltpu.SemaphoreType.DMA((N_HOPS,)),
            pltpu.SemaphoreType.DMA((N_HOPS,)),
            pltpu.SemaphoreType.DMA((N_HOPS,)),
        ],
        compiler_params=pltpu.CompilerParams(
            collective_id=0, vmem_limit_bytes=56 * 1024 * 1024
        ),
    )(Qr, KA, KB, VA, VB)
    return out


def kernel(Q, K, V):
    b, s, h, d = Q.shape
    BH = b * h
    H2 = s // 2

    def to_bhsd(x):
        return x.transpose(0, 2, 1, 3).reshape(BH, x.shape[1], d)

    Qr = to_bhsd(Q)
    KA = to_bhsd(K[:, :H2])
    KB = to_bhsd(K[:, H2:])
    VA = to_bhsd(V[:, :H2])
    VB = to_bhsd(V[:, H2:])

    out = _fused_agattn(Qr, KA, KB, VA, VB)
    return out.reshape(b, h, s, d).transpose(0, 2, 1, 3).astype(Q.dtype)


# device time: 1237686 ns/iter; 1.0675x vs baseline; 1.0078x over previous
import functools

import jax
import jax.numpy as jnp
from jax import lax
from jax.experimental import pallas as pl
from jax.experimental.pallas import tpu as pltpu

N_DEV = 8
N_HOPS = N_DEV - 1
MESH = pltpu.DeviceIdType.MESH
import os
DEBUG_NO_COMPUTE = os.environ.get("DEBUG_NO_COMPUTE") == "1"


def _fused_agattn(Qr, KA, KB, VA, VB):
    BH, S, D = Qr.shape
    H2 = S // 2
    scale = D ** -0.5
    f32 = jnp.float32

    NSPLIT = 2
    G = BH // NSPLIT

    def accum_unit(q_ref, k_s, v_s, l_ref, out_ref):
        for g in range(NSPLIT):
            sl = pl.ds(g * G, G)
            q = q_ref[sl]
            k = k_s[sl]
            v = v_s[sl]
            s_qk = lax.dot_general(
                q, k, (((2,), (2,)), ((0,), (0,))),
                preferred_element_type=f32,
            ) * scale
            p = jnp.exp(s_qk)
            l_ref[sl] = l_ref[sl] + jnp.sum(p, axis=-1)
            pv = lax.dot_general(
                p, v, (((2,), (1,)), ((0,), (0,))),
                preferred_element_type=f32,
            )
            out_ref[sl] = out_ref[sl] + pv

    def body(q_ref, ka_ref, kb_ref, va_ref, vb_ref,
             out_ref, kgA, kgB, vgA, vgB,
             kA_s, kB_s, vA_s, vB_s, l_ref,
             copy_sems, stage_sems,
             sA_k, rA_k, sA_v, rA_v, sB_k, rB_k, sB_v, rB_v):
        my = lax.axis_index("i")
        left = lax.rem(my - 1 + N_DEV, N_DEV)
        right = lax.rem(my + 1, N_DEV)

        barrier_sem = pltpu.get_barrier_semaphore()
        for nbr in (left, right):
            pl.semaphore_signal(barrier_sem, inc=1, device_id=(nbr,),
                                device_id_type=MESH)
        pl.semaphore_wait(barrier_sem, 2)

        own = [
            pltpu.make_async_copy(ka_ref, kgA.at[my], copy_sems.at[0]),
            pltpu.make_async_copy(kb_ref, kgB.at[my], copy_sems.at[1]),
            pltpu.make_async_copy(va_ref, vgA.at[my], copy_sems.at[2]),
            pltpu.make_async_copy(vb_ref, vgB.at[my], copy_sems.at[3]),
        ]
        stage = [
            pltpu.make_async_copy(ka_ref, kA_s, stage_sems.at[0]),
            pltpu.make_async_copy(kb_ref, kB_s, stage_sems.at[1]),
            pltpu.make_async_copy(va_ref, vA_s, stage_sems.at[2]),
            pltpu.make_async_copy(vb_ref, vB_s, stage_sems.at[3]),
        ]
        for c in own + stage:
            c.start()

        l_ref[...] = jnp.zeros((BH, S), f32)
        out_ref[...] = jnp.zeros((BH, S, D), f32)

        for c in own + stage:
            c.wait()

        for t in range(N_HOPS):
            cR = lax.rem(my - t + N_DEV, N_DEV)
            cL = lax.rem(my + t + N_DEV, N_DEV)
            rdmas = [
                pltpu.make_async_remote_copy(
                    src_ref=kgA.at[cR], dst_ref=kgA.at[cR],
                    send_sem=sA_k.at[t], recv_sem=rA_k.at[t],
                    device_id=(right,), device_id_type=MESH),
                pltpu.make_async_remote_copy(
                    src_ref=vgA.at[cR], dst_ref=vgA.at[cR],
                    send_sem=sA_v.at[t], recv_sem=rA_v.at[t],
                    device_id=(right,), device_id_type=MESH),
                pltpu.make_async_remote_copy(
                    src_ref=kgB.at[cL], dst_ref=kgB.at[cL],
                    send_sem=sB_k.at[t], recv_sem=rB_k.at[t],
                    device_id=(left,), device_id_type=MESH),
                pltpu.make_async_remote_copy(
                    src_ref=vgB.at[cL], dst_ref=vgB.at[cL],
                    send_sem=sB_v.at[t], recv_sem=rB_v.at[t],
                    device_id=(left,), device_id_type=MESH),
            ]
            for r in rdmas:
                r.start()

            if not DEBUG_NO_COMPUTE:
                accum_unit(q_ref, kA_s, vA_s, l_ref, out_ref)
                accum_unit(q_ref, kB_s, vB_s, l_ref, out_ref)

            for r in rdmas:
                r.wait()

            aR = lax.rem(my - t - 1 + N_DEV, N_DEV)
            aL = lax.rem(my + t + 1 + N_DEV, N_DEV)
            stage = [
                pltpu.make_async_copy(kgA.at[aR], kA_s, stage_sems.at[0]),
                pltpu.make_async_copy(vgA.at[aR], vA_s, stage_sems.at[1]),
                pltpu.make_async_copy(kgB.at[aL], kB_s, stage_sems.at[2]),
                pltpu.make_async_copy(vgB.at[aL], vB_s, stage_sems.at[3]),
            ]
            for c in stage:
                c.start()
            for c in stage:
                c.wait()

        if not DEBUG_NO_COMPUTE:
            accum_unit(q_ref, kA_s, vA_s, l_ref, out_ref)
            accum_unit(q_ref, kB_s, vB_s, l_ref, out_ref)

            out_ref[...] = out_ref[...] / l_ref[...][:, :, None]

        @functools.partial(pl.run_scoped,
                           exit_sem=pltpu.SemaphoreType.REGULAR)
        def _(exit_sem):
            for nbr in (left, right):
                pl.semaphore_signal(exit_sem, inc=1, device_id=(nbr,),
                                    device_id_type=MESH)
            pl.semaphore_wait(exit_sem, 2)

    hbm = pl.BlockSpec(memory_space=pltpu.MemorySpace.HBM)
    vmem = pl.BlockSpec(memory_space=pltpu.MemorySpace.VMEM)
    half = jax.ShapeDtypeStruct((N_DEV, BH, H2, D), f32)
    out, _, _, _, _ = pl.pallas_call(
        body,
        out_shape=(
            jax.ShapeDtypeStruct((BH, S, D), f32),
            half, half, half, half,
        ),
        in_specs=[vmem, hbm, hbm, hbm, hbm],
        out_specs=(vmem, hbm, hbm, hbm, hbm),
        scratch_shapes=[
            pltpu.VMEM((BH, H2, D), f32),
            pltpu.VMEM((BH, H2, D), f32),
            pltpu.VMEM((BH, H2, D), f32),
            pltpu.VMEM((BH, H2, D), f32),
            pltpu.VMEM((BH, S), f32),
            pltpu.SemaphoreType.DMA((4,)),
            pltpu.SemaphoreType.DMA((4,)),
            pltpu.SemaphoreType.DMA((N_HOPS,)),
            pltpu.SemaphoreType.DMA((N_HOPS,)),
            pltpu.SemaphoreType.DMA((N_HOPS,)),
            pltpu.SemaphoreType.DMA((N_HOPS,)),
            pltpu.SemaphoreType.DMA((N_HOPS,)),
            pltpu.SemaphoreType.DMA((N_HOPS,)),
            pltpu.SemaphoreType.DMA((N_HOPS,)),
            pltpu.SemaphoreType.DMA((N_HOPS,)),
        ],
        compiler_params=pltpu.CompilerParams(
            collective_id=0, vmem_limit_bytes=56 * 1024 * 1024
        ),
    )(Qr, KA, KB, VA, VB)
    return out


def kernel(Q, K, V):
    b, s, h, d = Q.shape
    BH = b * h
    H2 = s // 2

    def to_bhsd(x):
        return x.transpose(0, 2, 1, 3).reshape(BH, x.shape[1], d)

    Qr = to_bhsd(Q)
    KA = to_bhsd(K[:, :H2])
    KB = to_bhsd(K[:, H2:])
    VA = to_bhsd(V[:, :H2])
    VB = to_bhsd(V[:, H2:])

    out = _fused_agattn(Qr, KA, KB, VA, VB)
    return out.reshape(b, h, s, d).transpose(0, 2, 1, 3).astype(Q.dtype)


# device time: 650932 ns/iter; 2.0298x vs baseline; 1.9014x over previous
import functools
import os

import jax
import jax.numpy as jnp
from jax import lax
from jax.experimental import pallas as pl
from jax.experimental.pallas import tpu as pltpu

N_DEV = 8
N_HOPS = N_DEV - 1
MESH = pltpu.DeviceIdType.MESH
DEBUG_NO_COMPUTE = os.environ.get("DEBUG_NO_COMPUTE") == "1"


def _fused_agattn(Qt, KA, KB, VA, VB, S):
    BH, D, _ = Qt.shape
    H2 = S // 2
    scale = D ** -0.5
    f32 = jnp.float32

    NSPLIT = 2
    G = BH // NSPLIT

    def accum_unit(q_ref, k_s, v_s, l_ref, out_ref):
        for g in range(NSPLIT):
            sl = pl.ds(g * G, G)
            q = q_ref[sl]
            k = k_s[sl]
            v = v_s[sl]
            s_qk = lax.dot_general(
                q, k, (((1,), (1,)), ((0,), (0,))),
                preferred_element_type=f32,
            ) * scale
            p = jnp.exp(s_qk)
            l_ref[sl] = l_ref[sl] + jnp.sum(p, axis=-1)
            pv = lax.dot_general(
                p, v, (((2,), (2,)), ((0,), (0,))),
                preferred_element_type=f32,
            )
            out_ref[sl] = out_ref[sl] + pv

    def body(q_ref, ka_ref, kb_ref, va_ref, vb_ref,
             out_ref, kgA, kgB, vgA, vgB,
             kA_s, kB_s, vA_s, vB_s, l_ref,
             copy_sems, stage_sems,
             sA_k, rA_k, sA_v, rA_v, sB_k, rB_k, sB_v, rB_v):
        my = lax.axis_index("i")
        left = lax.rem(my - 1 + N_DEV, N_DEV)
        right = lax.rem(my + 1, N_DEV)

        barrier_sem = pltpu.get_barrier_semaphore()
        for nbr in (left, right):
            pl.semaphore_signal(barrier_sem, inc=1, device_id=(nbr,),
                                device_id_type=MESH)
        pl.semaphore_wait(barrier_sem, 2)

        own = [
            pltpu.make_async_copy(ka_ref, kgA.at[my], copy_sems.at[0]),
            pltpu.make_async_copy(kb_ref, kgB.at[my], copy_sems.at[1]),
            pltpu.make_async_copy(va_ref, vgA.at[my], copy_sems.at[2]),
            pltpu.make_async_copy(vb_ref, vgB.at[my], copy_sems.at[3]),
        ]
        stage = [
            pltpu.make_async_copy(ka_ref, kA_s, stage_sems.at[0]),
            pltpu.make_async_copy(kb_ref, kB_s, stage_sems.at[1]),
            pltpu.make_async_copy(va_ref, vA_s, stage_sems.at[2]),
            pltpu.make_async_copy(vb_ref, vB_s, stage_sems.at[3]),
        ]
        for c in own + stage:
            c.start()

        l_ref[...] = jnp.zeros((BH, S), f32)
        out_ref[...] = jnp.zeros((BH, S, D), f32)

        for c in own + stage:
            c.wait()

        for t in range(N_HOPS):
            cR = lax.rem(my - t + N_DEV, N_DEV)
            cL = lax.rem(my + t + N_DEV, N_DEV)
            rdmas = [
                pltpu.make_async_remote_copy(
                    src_ref=kgA.at[cR], dst_ref=kgA.at[cR],
                    send_sem=sA_k.at[t], recv_sem=rA_k.at[t],
                    device_id=(right,), device_id_type=MESH),
                pltpu.make_async_remote_copy(
                    src_ref=vgA.at[cR], dst_ref=vgA.at[cR],
                    send_sem=sA_v.at[t], recv_sem=rA_v.at[t],
                    device_id=(right,), device_id_type=MESH),
                pltpu.make_async_remote_copy(
                    src_ref=kgB.at[cL], dst_ref=kgB.at[cL],
                    send_sem=sB_k.at[t], recv_sem=rB_k.at[t],
                    device_id=(left,), device_id_type=MESH),
                pltpu.make_async_remote_copy(
                    src_ref=vgB.at[cL], dst_ref=vgB.at[cL],
                    send_sem=sB_v.at[t], recv_sem=rB_v.at[t],
                    device_id=(left,), device_id_type=MESH),
            ]
            for r in rdmas:
                r.start()

            if not DEBUG_NO_COMPUTE:
                accum_unit(q_ref, kA_s, vA_s, l_ref, out_ref)
                accum_unit(q_ref, kB_s, vB_s, l_ref, out_ref)

            for r in rdmas:
                r.wait()

            aR = lax.rem(my - t - 1 + N_DEV, N_DEV)
            aL = lax.rem(my + t + 1 + N_DEV, N_DEV)
            stage = [
                pltpu.make_async_copy(kgA.at[aR], kA_s, stage_sems.at[0]),
                pltpu.make_async_copy(vgA.at[aR], vA_s, stage_sems.at[1]),
                pltpu.make_async_copy(kgB.at[aL], kB_s, stage_sems.at[2]),
                pltpu.make_async_copy(vgB.at[aL], vB_s, stage_sems.at[3]),
            ]
            for c in stage:
                c.start()
            for c in stage:
                c.wait()

        if not DEBUG_NO_COMPUTE:
            accum_unit(q_ref, kA_s, vA_s, l_ref, out_ref)
            accum_unit(q_ref, kB_s, vB_s, l_ref, out_ref)

            out_ref[...] = out_ref[...] / l_ref[...][:, :, None]

        @functools.partial(pl.run_scoped,
                           exit_sem=pltpu.SemaphoreType.REGULAR)
        def _(exit_sem):
            for nbr in (left, right):
                pl.semaphore_signal(exit_sem, inc=1, device_id=(nbr,),
                                    device_id_type=MESH)
            pl.semaphore_wait(exit_sem, 2)

    hbm = pl.BlockSpec(memory_space=pltpu.MemorySpace.HBM)
    vmem = pl.BlockSpec(memory_space=pltpu.MemorySpace.VMEM)
    half = jax.ShapeDtypeStruct((N_DEV, BH, D, H2), jnp.float32)
    out, _, _, _, _ = pl.pallas_call(
        body,
        out_shape=(
            jax.ShapeDtypeStruct((BH, S, D), jnp.float32),
            half, half, half, half,
        ),
        in_specs=[vmem, hbm, hbm, hbm, hbm],
        out_specs=(vmem, hbm, hbm, hbm, hbm),
        scratch_shapes=[
            pltpu.VMEM((BH, D, H2), jnp.float32),
            pltpu.VMEM((BH, D, H2), jnp.float32),
            pltpu.VMEM((BH, D, H2), jnp.float32),
            pltpu.VMEM((BH, D, H2), jnp.float32),
            pltpu.VMEM((BH, S), jnp.float32),
            pltpu.SemaphoreType.DMA((4,)),
            pltpu.SemaphoreType.DMA((4,)),
            pltpu.SemaphoreType.DMA((N_HOPS,)),
            pltpu.SemaphoreType.DMA((N_HOPS,)),
            pltpu.SemaphoreType.DMA((N_HOPS,)),
            pltpu.SemaphoreType.DMA((N_HOPS,)),
            pltpu.SemaphoreType.DMA((N_HOPS,)),
            pltpu.SemaphoreType.DMA((N_HOPS,)),
            pltpu.SemaphoreType.DMA((N_HOPS,)),
            pltpu.SemaphoreType.DMA((N_HOPS,)),
        ],
        compiler_params=pltpu.CompilerParams(
            collective_id=0, vmem_limit_bytes=56 * 1024 * 1024
        ),
    )(Qt, KA, KB, VA, VB)
    return out


def kernel(Q, K, V):
    b, s, h, d = Q.shape
    BH = b * h
    H2 = s // 2

    def to_dmajor(x):
        return x.transpose(0, 2, 3, 1).reshape(BH, d, x.shape[1])

    Qt = to_dmajor(Q)
    KA = to_dmajor(K[:, :H2])
    KB = to_dmajor(K[:, H2:])
    VA = to_dmajor(V[:, :H2])
    VB = to_dmajor(V[:, H2:])

    out = _fused_agattn(Qt, KA, KB, VA, VB, s)
    return out.reshape(b, h, s, d).transpose(0, 2, 1, 3).astype(Q.dtype)


# device time: 647915 ns/iter; 2.0393x vs baseline; 1.0047x over previous
import functools
import os

import jax
import jax.numpy as jnp
from jax import lax
from jax.experimental import pallas as pl
from jax.experimental.pallas import tpu as pltpu

N_DEV = 8
N_HOPS = N_DEV - 1
MESH = pltpu.DeviceIdType.MESH
DEBUG_NO_COMPUTE = os.environ.get("DEBUG_NO_COMPUTE") == "1"


def _fused_agattn(Qt, KA, KB, VA, VB, S):
    BH, D, _ = Qt.shape
    H2 = S // 2
    scale = D ** -0.5
    f32 = jnp.float32

    NSPLIT = 2
    G = BH // NSPLIT

    def accum_unit(q_ref, k_s, v_s, l_ref, out_ref):
        for g in range(NSPLIT):
            sl = pl.ds(g * G, G)
            q = q_ref[sl]
            k = k_s[sl]
            v = v_s[sl]
            s_qk = lax.dot_general(
                q, k, (((1,), (1,)), ((0,), (0,))),
                preferred_element_type=f32,
            ) * scale
            p = jnp.exp(s_qk)
            l_ref[sl] = l_ref[sl] + jnp.sum(p, axis=-1)
            pv = lax.dot_general(
                p, v, (((2,), (2,)), ((0,), (0,))),
                preferred_element_type=f32,
            )
            out_ref[sl] = out_ref[sl] + pv

    def body(q_ref, ka_ref, kb_ref, va_ref, vb_ref,
             out_ref, kgA, kgB, vgA, vgB,
             kA_s, kB_s, vA_s, vB_s, l_ref,
             copy_sems, stage_sems,
             sA_k, rA_k, sA_v, rA_v, sB_k, rB_k, sB_v, rB_v):
        my = lax.axis_index("i")

        def ring_id(p):
            p = lax.rem(p + 2 * N_DEV, N_DEV)
            return jnp.where(p < 4, p, 11 - p)

        pos = ring_id(my)
        left = ring_id(pos - 1)
        right = ring_id(pos + 1)

        barrier_sem = pltpu.get_barrier_semaphore()
        for nbr in (left, right):
            pl.semaphore_signal(barrier_sem, inc=1, device_id=(nbr,),
                                device_id_type=MESH)
        pl.semaphore_wait(barrier_sem, 2)

        own = [
            pltpu.make_async_copy(ka_ref, kgA.at[my], copy_sems.at[0]),
            pltpu.make_async_copy(kb_ref, kgB.at[my], copy_sems.at[1]),
            pltpu.make_async_copy(va_ref, vgA.at[my], copy_sems.at[2]),
            pltpu.make_async_copy(vb_ref, vgB.at[my], copy_sems.at[3]),
        ]
        stage = [
            pltpu.make_async_copy(ka_ref, kA_s, stage_sems.at[0]),
            pltpu.make_async_copy(kb_ref, kB_s, stage_sems.at[1]),
            pltpu.make_async_copy(va_ref, vA_s, stage_sems.at[2]),
            pltpu.make_async_copy(vb_ref, vB_s, stage_sems.at[3]),
        ]
        for c in own + stage:
            c.start()

        l_ref[...] = jnp.zeros((BH, S), f32)
        out_ref[...] = jnp.zeros((BH, S, D), f32)

        for c in own + stage:
            c.wait()

        for t in range(N_HOPS):
            cR = ring_id(pos - t)
            cL = ring_id(pos + t)
            rdmas = [
                pltpu.make_async_remote_copy(
                    src_ref=kgA.at[cR], dst_ref=kgA.at[cR],
                    send_sem=sA_k.at[t], recv_sem=rA_k.at[t],
                    device_id=(right,), device_id_type=MESH),
                pltpu.make_async_remote_copy(
                    src_ref=vgA.at[cR], dst_ref=vgA.at[cR],
                    send_sem=sA_v.at[t], recv_sem=rA_v.at[t],
                    device_id=(right,), device_id_type=MESH),
                pltpu.make_async_remote_copy(
                    src_ref=kgB.at[cL], dst_ref=kgB.at[cL],
                    send_sem=sB_k.at[t], recv_sem=rB_k.at[t],
                    device_id=(left,), device_id_type=MESH),
                pltpu.make_async_remote_copy(
                    src_ref=vgB.at[cL], dst_ref=vgB.at[cL],
                    send_sem=sB_v.at[t], recv_sem=rB_v.at[t],
                    device_id=(left,), device_id_type=MESH),
            ]
            for r in rdmas:
                r.start()

            if not DEBUG_NO_COMPUTE:
                accum_unit(q_ref, kA_s, vA_s, l_ref, out_ref)
                accum_unit(q_ref, kB_s, vB_s, l_ref, out_ref)

            for r in rdmas:
                r.wait()

            aR = ring_id(pos - t - 1)
            aL = ring_id(pos + t + 1)
            stage = [
                pltpu.make_async_copy(kgA.at[aR], kA_s, stage_sems.at[0]),
                pltpu.make_async_copy(vgA.at[aR], vA_s, stage_sems.at[1]),
                pltpu.make_async_copy(kgB.at[aL], kB_s, stage_sems.at[2]),
                pltpu.make_async_copy(vgB.at[aL], vB_s, stage_sems.at[3]),
            ]
            for c in stage:
                c.start()
            for c in stage:
                c.wait()

        if not DEBUG_NO_COMPUTE:
            accum_unit(q_ref, kA_s, vA_s, l_ref, out_ref)
            accum_unit(q_ref, kB_s, vB_s, l_ref, out_ref)

            out_ref[...] = out_ref[...] / l_ref[...][:, :, None]

        @functools.partial(pl.run_scoped,
                           exit_sem=pltpu.SemaphoreType.REGULAR)
        def _(exit_sem):
            for nbr in (left, right):
                pl.semaphore_signal(exit_sem, inc=1, device_id=(nbr,),
                                    device_id_type=MESH)
            pl.semaphore_wait(exit_sem, 2)

    hbm = pl.BlockSpec(memory_space=pltpu.MemorySpace.HBM)
    vmem = pl.BlockSpec(memory_space=pltpu.MemorySpace.VMEM)
    half = jax.ShapeDtypeStruct((N_DEV, BH, D, H2), jnp.float32)
    out, _, _, _, _ = pl.pallas_call(
        body,
        out_shape=(
            jax.ShapeDtypeStruct((BH, S, D), jnp.float32),
            half, half, half, half,
        ),
        in_specs=[vmem, hbm, hbm, hbm, hbm],
        out_specs=(vmem, hbm, hbm, hbm, hbm),
        scratch_shapes=[
            pltpu.VMEM((BH, D, H2), jnp.float32),
            pltpu.VMEM((BH, D, H2), jnp.float32),
            pltpu.VMEM((BH, D, H2), jnp.float32),
            pltpu.VMEM((BH, D, H2), jnp.float32),
            pltpu.VMEM((BH, S), jnp.float32),
            pltpu.SemaphoreType.DMA((4,)),
            pltpu.SemaphoreType.DMA((4,)),
            pltpu.SemaphoreType.DMA((N_HOPS,)),
            pltpu.SemaphoreType.DMA((N_HOPS,)),
            pltpu.SemaphoreType.DMA((N_HOPS,)),
            pltpu.SemaphoreType.DMA((N_HOPS,)),
            pltpu.SemaphoreType.DMA((N_HOPS,)),
            pltpu.SemaphoreType.DMA((N_HOPS,)),
            pltpu.SemaphoreType.DMA((N_HOPS,)),
            pltpu.SemaphoreType.DMA((N_HOPS,)),
        ],
        compiler_params=pltpu.CompilerParams(
            collective_id=0, vmem_limit_bytes=56 * 1024 * 1024
        ),
    )(Qt, KA, KB, VA, VB)
    return out


def kernel(Q, K, V):
    b, s, h, d = Q.shape
    BH = b * h
    H2 = s // 2

    def to_dmajor(x):
        return x.transpose(0, 2, 3, 1).reshape(BH, d, x.shape[1])

    Qt = to_dmajor(Q)
    KA = to_dmajor(K[:, :H2])
    KB = to_dmajor(K[:, H2:])
    VA = to_dmajor(V[:, :H2])
    VB = to_dmajor(V[:, H2:])

    out = _fused_agattn(Qt, KA, KB, VA, VB, s)
    return out.reshape(b, h, s, d).transpose(0, 2, 1, 3).astype(Q.dtype)


# device time: 394272 ns/iter; 3.3511x vs baseline; 1.6433x over previous
import functools
import os

import jax
import jax.numpy as jnp
from jax import lax
from jax.experimental import pallas as pl
from jax.experimental.pallas import tpu as pltpu

N_DEV = 8
N_HOPS = N_DEV - 1
MESH = pltpu.DeviceIdType.MESH
DEBUG_NO_COMPUTE = os.environ.get("DEBUG_NO_COMPUTE") == "1"


def _fused_agattn(Qt, KA, KB, VA, VB, S):
    BH, D, _ = Qt.shape
    H2 = S // 2
    scale = D ** -0.5
    f32 = jnp.float32

    NSPLIT = 2
    G = BH // NSPLIT

    def accum_unit(q_ref, k_s, v_s, l_ref, out_ref):
        for g in range(NSPLIT):
            sl = pl.ds(g * G, G)
            q = q_ref[sl]
            k = k_s[sl]
            v = v_s[sl]
            s_qk = lax.dot_general(
                q, k, (((1,), (1,)), ((0,), (0,))),
                preferred_element_type=f32,
            ) * scale
            p = jnp.exp(s_qk)
            l_ref[sl] = l_ref[sl] + jnp.sum(p, axis=-1)
            pv = lax.dot_general(
                p, v, (((2,), (2,)), ((0,), (0,))),
                preferred_element_type=f32,
            )
            out_ref[sl] = out_ref[sl] + pv

    def body(q_ref, ka_ref, kb_ref, va_ref, vb_ref,
             out_ref, kgA, kgB, vgA, vgB,
             kA_s, kB_s, vA_s, vB_s, l_ref,
             stage_sems,
             sA_k, rA_k, sA_v, rA_v, sB_k, rB_k, sB_v, rB_v):
        my = lax.axis_index("i")

        def ring_id(p):
            p = lax.rem(p + 2 * N_DEV, N_DEV)
            return jnp.where(p < 4, p, 11 - p)

        pos = ring_id(my)
        left = ring_id(pos - 1)
        right = ring_id(pos + 1)

        barrier_sem = pltpu.get_barrier_semaphore()
        for nbr in (left, right):
            pl.semaphore_signal(barrier_sem, inc=1, device_id=(nbr,),
                                device_id_type=MESH)
        pl.semaphore_wait(barrier_sem, 2)

        stage = [
            pltpu.make_async_copy(ka_ref, kA_s, stage_sems.at[0]),
            pltpu.make_async_copy(kb_ref, kB_s, stage_sems.at[1]),
            pltpu.make_async_copy(va_ref, vA_s, stage_sems.at[2]),
            pltpu.make_async_copy(vb_ref, vB_s, stage_sems.at[3]),
        ]
        for c in stage:
            c.start()

        l_ref[...] = jnp.zeros((BH, S), f32)
        out_ref[...] = jnp.zeros((BH, S, D), f32)

        for c in stage:
            c.wait()

        for t in range(N_HOPS):
            cR = ring_id(pos - t)
            cL = ring_id(pos + t)
            rdmas = [
                pltpu.make_async_remote_copy(
                    src_ref=kA_s, dst_ref=kgA.at[cR],
                    send_sem=sA_k.at[t], recv_sem=rA_k.at[t],
                    device_id=(right,), device_id_type=MESH),
                pltpu.make_async_remote_copy(
                    src_ref=vA_s, dst_ref=vgA.at[cR],
                    send_sem=sA_v.at[t], recv_sem=rA_v.at[t],
                    device_id=(right,), device_id_type=MESH),
                pltpu.make_async_remote_copy(
                    src_ref=kB_s, dst_ref=kgB.at[cL],
                    send_sem=sB_k.at[t], recv_sem=rB_k.at[t],
                    device_id=(left,), device_id_type=MESH),
                pltpu.make_async_remote_copy(
                    src_ref=vB_s, dst_ref=vgB.at[cL],
                    send_sem=sB_v.at[t], recv_sem=rB_v.at[t],
                    device_id=(left,), device_id_type=MESH),
            ]
            for r in rdmas:
                r.start()

            if not DEBUG_NO_COMPUTE:
                accum_unit(q_ref, kA_s, vA_s, l_ref, out_ref)
                accum_unit(q_ref, kB_s, vB_s, l_ref, out_ref)

            for r in rdmas:
                r.wait()

            aR = ring_id(pos - t - 1)
            aL = ring_id(pos + t + 1)
            stage = [
                pltpu.make_async_copy(kgA.at[aR], kA_s, stage_sems.at[0]),
                pltpu.make_async_copy(vgA.at[aR], vA_s, stage_sems.at[1]),
                pltpu.make_async_copy(kgB.at[aL], kB_s, stage_sems.at[2]),
                pltpu.make_async_copy(vgB.at[aL], vB_s, stage_sems.at[3]),
            ]
            for c in stage:
                c.start()
            for c in stage:
                c.wait()

        if not DEBUG_NO_COMPUTE:
            accum_unit(q_ref, kA_s, vA_s, l_ref, out_ref)
            accum_unit(q_ref, kB_s, vB_s, l_ref, out_ref)

            out_ref[...] = out_ref[...] / l_ref[...][:, :, None]

        @functools.partial(pl.run_scoped,
                           exit_sem=pltpu.SemaphoreType.REGULAR)
        def _(exit_sem):
            for nbr in (left, right):
                pl.semaphore_signal(exit_sem, inc=1, device_id=(nbr,),
                                    device_id_type=MESH)
            pl.semaphore_wait(exit_sem, 2)

    hbm = pl.BlockSpec(memory_space=pltpu.MemorySpace.HBM)
    vmem = pl.BlockSpec(memory_space=pltpu.MemorySpace.VMEM)
    half = jax.ShapeDtypeStruct((N_DEV, BH, D, H2), jnp.float32)
    out, _, _, _, _ = pl.pallas_call(
        body,
        out_shape=(
            jax.ShapeDtypeStruct((BH, S, D), jnp.float32),
            half, half, half, half,
        ),
        in_specs=[vmem, hbm, hbm, hbm, hbm],
        out_specs=(vmem, hbm, hbm, hbm, hbm),
        scratch_shapes=[
            pltpu.VMEM((BH, D, H2), jnp.float32),
            pltpu.VMEM((BH, D, H2), jnp.float32),
            pltpu.VMEM((BH, D, H2), jnp.float32),
            pltpu.VMEM((BH, D, H2), jnp.float32),
            pltpu.VMEM((BH, S), jnp.float32),
            pltpu.SemaphoreType.DMA((4,)),
            pltpu.SemaphoreType.DMA((N_HOPS,)),
            pltpu.SemaphoreType.DMA((N_HOPS,)),
            pltpu.SemaphoreType.DMA((N_HOPS,)),
            pltpu.SemaphoreType.DMA((N_HOPS,)),
            pltpu.SemaphoreType.DMA((N_HOPS,)),
            pltpu.SemaphoreType.DMA((N_HOPS,)),
            pltpu.SemaphoreType.DMA((N_HOPS,)),
            pltpu.SemaphoreType.DMA((N_HOPS,)),
        ],
        compiler_params=pltpu.CompilerParams(
            collective_id=0, vmem_limit_bytes=56 * 1024 * 1024
        ),
    )(Qt, KA, KB, VA, VB)
    return out


def kernel(Q, K, V):
    b, s, h, d = Q.shape
    BH = b * h
    H2 = s // 2

    def to_dmajor(x):
        return x.transpose(0, 2, 3, 1).reshape(BH, d, x.shape[1])

    Qt = to_dmajor(Q)
    KA = to_dmajor(K[:, :H2])
    KB = to_dmajor(K[:, H2:])
    VA = to_dmajor(V[:, :H2])
    VB = to_dmajor(V[:, H2:])

    out = _fused_agattn(Qt, KA, KB, VA, VB, s)
    return out.reshape(b, h, s, d).transpose(0, 2, 1, 3).astype(Q.dtype)


# device time: 373402 ns/iter; 3.5384x vs baseline; 1.0559x over previous
import functools
import os

import jax
import jax.numpy as jnp
from jax import lax
from jax.experimental import pallas as pl
from jax.experimental.pallas import tpu as pltpu

N_DEV = 8
N_HOPS = N_DEV - 1
MESH = pltpu.DeviceIdType.MESH
DEBUG_NO_COMPUTE = os.environ.get("DEBUG_NO_COMPUTE") == "1"


def _fused_agattn(Qt, KA, KB, VA, VB, S):
    BH, D, _ = Qt.shape
    H2 = S // 2
    scale = D ** -0.5
    f32 = jnp.float32

    NSPLIT = 2
    G = BH // NSPLIT

    def accum_unit(q_ref, k_set, v_set, sl_set, l_ref, out_ref):
        for g in range(NSPLIT):
            sl = pl.ds(g * G, G)
            q = q_ref[sl]
            k = k_set[sl_set, sl]
            v = v_set[sl_set, sl]
            s_qk = lax.dot_general(
                q, k, (((1,), (1,)), ((0,), (0,))),
                preferred_element_type=f32,
            ) * scale
            p = jnp.exp(s_qk)
            l_ref[sl] = l_ref[sl] + jnp.sum(p, axis=-1)
            pv = lax.dot_general(
                p, v, (((2,), (2,)), ((0,), (0,))),
                preferred_element_type=f32,
            )
            out_ref[sl] = out_ref[sl] + pv

    def body(q_ref, ka_ref, kb_ref, va_ref, vb_ref, out_ref,
             kA_set, kB_set, vA_set, vB_set, l_ref,
             stage_sems, creditA, creditB,
             sA_k, rA_k, sA_v, rA_v, sB_k, rB_k, sB_v, rB_v):
        my = lax.axis_index("i")

        def ring_id(p):
            p = lax.rem(p + 2 * N_DEV, N_DEV)
            return jnp.where(p < 4, p, 11 - p)

        pos = ring_id(my)
        left = ring_id(pos - 1)
        right = ring_id(pos + 1)

        barrier_sem = pltpu.get_barrier_semaphore()
        for nbr in (left, right):
            pl.semaphore_signal(barrier_sem, inc=1, device_id=(nbr,),
                                device_id_type=MESH)
        pl.semaphore_wait(barrier_sem, 2)

        stage = [
            pltpu.make_async_copy(ka_ref, kA_set.at[0], stage_sems.at[0]),
            pltpu.make_async_copy(kb_ref, kB_set.at[0], stage_sems.at[1]),
            pltpu.make_async_copy(va_ref, vA_set.at[0], stage_sems.at[2]),
            pltpu.make_async_copy(vb_ref, vB_set.at[0], stage_sems.at[3]),
        ]
        for c in stage:
            c.start()

        l_ref[...] = jnp.zeros((BH, S), f32)
        out_ref[...] = jnp.zeros((BH, S, D), f32)

        for c in stage:
            c.wait()

        for t in range(N_HOPS):
            cur, nxt = t % 2, (t + 1) % 2
            if t >= 1:
                pl.semaphore_wait(creditA, 1)
                pl.semaphore_wait(creditB, 1)
            rdmas = [
                pltpu.make_async_remote_copy(
                    src_ref=kA_set.at[cur], dst_ref=kA_set.at[nxt],
                    send_sem=sA_k.at[t], recv_sem=rA_k.at[t],
                    device_id=(right,), device_id_type=MESH),
                pltpu.make_async_remote_copy(
                    src_ref=vA_set.at[cur], dst_ref=vA_set.at[nxt],
                    send_sem=sA_v.at[t], recv_sem=rA_v.at[t],
                    device_id=(right,), device_id_type=MESH),
                pltpu.make_async_remote_copy(
                    src_ref=kB_set.at[cur], dst_ref=kB_set.at[nxt],
                    send_sem=sB_k.at[t], recv_sem=rB_k.at[t],
                    device_id=(left,), device_id_type=MESH),
                pltpu.make_async_remote_copy(
                    src_ref=vB_set.at[cur], dst_ref=vB_set.at[nxt],
                    send_sem=sB_v.at[t], recv_sem=rB_v.at[t],
                    device_id=(left,), device_id_type=MESH),
            ]
            for r in rdmas:
                r.start()

            if not DEBUG_NO_COMPUTE:
                accum_unit(q_ref, kA_set, vA_set, cur, l_ref, out_ref)
                accum_unit(q_ref, kB_set, vB_set, cur, l_ref, out_ref)

            for r in rdmas:
                r.wait()

            if t < N_HOPS - 1:
                pl.semaphore_signal(creditA, inc=1, device_id=(left,),
                                    device_id_type=MESH)
                pl.semaphore_signal(creditB, inc=1, device_id=(right,),
                                    device_id_type=MESH)

        if not DEBUG_NO_COMPUTE:
            accum_unit(q_ref, kA_set, vA_set, N_HOPS % 2, l_ref, out_ref)
            accum_unit(q_ref, kB_set, vB_set, N_HOPS % 2, l_ref, out_ref)

            out_ref[...] = out_ref[...] / l_ref[...][:, :, None]

        @functools.partial(pl.run_scoped,
                           exit_sem=pltpu.SemaphoreType.REGULAR)
        def _(exit_sem):
            for nbr in (left, right):
                pl.semaphore_signal(exit_sem, inc=1, device_id=(nbr,),
                                    device_id_type=MESH)
            pl.semaphore_wait(exit_sem, 2)

    hbm = pl.BlockSpec(memory_space=pltpu.MemorySpace.HBM)
    vmem = pl.BlockSpec(memory_space=pltpu.MemorySpace.VMEM)
    out = pl.pallas_call(
        body,
        out_shape=jax.ShapeDtypeStruct((BH, S, D), jnp.float32),
        in_specs=[vmem, hbm, hbm, hbm, hbm],
        out_specs=vmem,
        scratch_shapes=[
            pltpu.VMEM((2, BH, D, H2), jnp.float32),
            pltpu.VMEM((2, BH, D, H2), jnp.float32),
            pltpu.VMEM((2, BH, D, H2), jnp.float32),
            pltpu.VMEM((2, BH, D, H2), jnp.float32),
            pltpu.VMEM((BH, S), jnp.float32),
            pltpu.SemaphoreType.DMA((4,)),
            pltpu.SemaphoreType.REGULAR,
            pltpu.SemaphoreType.REGULAR,
            pltpu.SemaphoreType.DMA((N_HOPS,)),
            pltpu.SemaphoreType.DMA((N_HOPS,)),
            pltpu.SemaphoreType.DMA((N_HOPS,)),
            pltpu.SemaphoreType.DMA((N_HOPS,)),
            pltpu.SemaphoreType.DMA((N_HOPS,)),
            pltpu.SemaphoreType.DMA((N_HOPS,)),
            pltpu.SemaphoreType.DMA((N_HOPS,)),
            pltpu.SemaphoreType.DMA((N_HOPS,)),
        ],
        compiler_params=pltpu.CompilerParams(
            collective_id=0, vmem_limit_bytes=60 * 1024 * 1024
        ),
    )(Qt, KA, KB, VA, VB)
    return out


def kernel(Q, K, V):
    b, s, h, d = Q.shape
    BH = b * h
    H2 = s // 2

    def to_dmajor(x):
        return x.transpose(0, 2, 3, 1).reshape(BH, d, x.shape[1])

    Qt = to_dmajor(Q)
    KA = to_dmajor(K[:, :H2])
    KB = to_dmajor(K[:, H2:])
    VA = to_dmajor(V[:, :H2])
    VB = to_dmajor(V[:, H2:])

    out = _fused_agattn(Qt, KA, KB, VA, VB, s)
    return out.reshape(b, h, s, d).transpose(0, 2, 1, 3).astype(Q.dtype)


# device time: 208780 ns/iter; 6.3285x vs baseline; 1.7885x over previous
import functools
import os

import jax
import jax.numpy as jnp
from jax import lax
from jax.experimental import pallas as pl
from jax.experimental.pallas import tpu as pltpu

N_DEV = 8
N_HOPS = N_DEV - 1
MESH = pltpu.DeviceIdType.MESH
DEBUG_NO_COMPUTE = os.environ.get("DEBUG_NO_COMPUTE") == "1"


def _fused_agattn(Qt, KA, KB, VA, VB, S):
    BH, D, _ = Qt.shape
    H2 = S // 2
    scale = D ** -0.5
    f32 = jnp.float32
    bf16 = jnp.bfloat16

    NSPLIT = 2
    G = BH // NSPLIT

    def accum_unit(q_ref, k_set, v_set, sl_set, l_ref, out_ref):
        for g in range(NSPLIT):
            sl = pl.ds(g * G, G)
            q = q_ref[sl]
            k = k_set[sl_set, sl]
            v = v_set[sl_set, sl]
            s_qk = lax.dot_general(
                q, k, (((1,), (1,)), ((0,), (0,))),
                preferred_element_type=f32,
            ) * scale
            p = jnp.exp(s_qk)
            l_ref[sl] = l_ref[sl] + jnp.sum(p, axis=-1)
            pv = lax.dot_general(
                p.astype(bf16), v, (((2,), (2,)), ((0,), (0,))),
                preferred_element_type=f32,
            )
            out_ref[sl] = out_ref[sl] + pv

    def body(q_ref, ka_ref, kb_ref, va_ref, vb_ref, out_ref,
             kA_set, kB_set, vA_set, vB_set, l_ref,
             stage_sems, creditA, creditB,
             sA_k, rA_k, sA_v, rA_v, sB_k, rB_k, sB_v, rB_v):
        my = lax.axis_index("i")

        def ring_id(p):
            p = lax.rem(p + 2 * N_DEV, N_DEV)
            return jnp.where(p < 4, p, 11 - p)

        pos = ring_id(my)
        left = ring_id(pos - 1)
        right = ring_id(pos + 1)

        barrier_sem = pltpu.get_barrier_semaphore()
        for nbr in (left, right):
            pl.semaphore_signal(barrier_sem, inc=1, device_id=(nbr,),
                                device_id_type=MESH)
        pl.semaphore_wait(barrier_sem, 2)

        stage = [
            pltpu.make_async_copy(ka_ref, kA_set.at[0], stage_sems.at[0]),
            pltpu.make_async_copy(kb_ref, kB_set.at[0], stage_sems.at[1]),
            pltpu.make_async_copy(va_ref, vA_set.at[0], stage_sems.at[2]),
            pltpu.make_async_copy(vb_ref, vB_set.at[0], stage_sems.at[3]),
        ]
        for c in stage:
            c.start()

        l_ref[...] = jnp.zeros((BH, S), f32)
        out_ref[...] = jnp.zeros((BH, S, D), f32)

        for c in stage:
            c.wait()

        for t in range(N_HOPS):
            cur, nxt = t % 2, (t + 1) % 2
            if t >= 1:
                pl.semaphore_wait(creditA, 1)
                pl.semaphore_wait(creditB, 1)
            rdmas = [
                pltpu.make_async_remote_copy(
                    src_ref=kA_set.at[cur], dst_ref=kA_set.at[nxt],
                    send_sem=sA_k.at[t], recv_sem=rA_k.at[t],
                    device_id=(right,), device_id_type=MESH),
                pltpu.make_async_remote_copy(
                    src_ref=vA_set.at[cur], dst_ref=vA_set.at[nxt],
                    send_sem=sA_v.at[t], recv_sem=rA_v.at[t],
                    device_id=(right,), device_id_type=MESH),
                pltpu.make_async_remote_copy(
                    src_ref=kB_set.at[cur], dst_ref=kB_set.at[nxt],
                    send_sem=sB_k.at[t], recv_sem=rB_k.at[t],
                    device_id=(left,), device_id_type=MESH),
                pltpu.make_async_remote_copy(
                    src_ref=vB_set.at[cur], dst_ref=vB_set.at[nxt],
                    send_sem=sB_v.at[t], recv_sem=rB_v.at[t],
                    device_id=(left,), device_id_type=MESH),
            ]
            for r in rdmas:
                r.start()

            if not DEBUG_NO_COMPUTE:
                accum_unit(q_ref, kA_set, vA_set, cur, l_ref, out_ref)
                accum_unit(q_ref, kB_set, vB_set, cur, l_ref, out_ref)

            for r in rdmas:
                r.wait_send()
            if t < N_HOPS - 1:
                pl.semaphore_signal(creditA, inc=1, device_id=(left,),
                                    device_id_type=MESH)
                pl.semaphore_signal(creditB, inc=1, device_id=(right,),
                                    device_id_type=MESH)
            for r in rdmas:
                r.wait_recv()

        if not DEBUG_NO_COMPUTE:
            accum_unit(q_ref, kA_set, vA_set, N_HOPS % 2, l_ref, out_ref)
            accum_unit(q_ref, kB_set, vB_set, N_HOPS % 2, l_ref, out_ref)

            out_ref[...] = out_ref[...] / l_ref[...][:, :, None]

        @functools.partial(pl.run_scoped,
                           exit_sem=pltpu.SemaphoreType.REGULAR)
        def _(exit_sem):
            for nbr in (left, right):
                pl.semaphore_signal(exit_sem, inc=1, device_id=(nbr,),
                                    device_id_type=MESH)
            pl.semaphore_wait(exit_sem, 2)

    hbm = pl.BlockSpec(memory_space=pltpu.MemorySpace.HBM)
    vmem = pl.BlockSpec(memory_space=pltpu.MemorySpace.VMEM)
    out = pl.pallas_call(
        body,
        out_shape=jax.ShapeDtypeStruct((BH, S, D), jnp.float32),
        in_specs=[vmem, hbm, hbm, hbm, hbm],
        out_specs=vmem,
        scratch_shapes=[
            pltpu.VMEM((2, BH, D, H2), jnp.bfloat16),
            pltpu.VMEM((2, BH, D, H2), jnp.bfloat16),
            pltpu.VMEM((2, BH, D, H2), jnp.bfloat16),
            pltpu.VMEM((2, BH, D, H2), jnp.bfloat16),
            pltpu.VMEM((BH, S), jnp.float32),
            pltpu.SemaphoreType.DMA((4,)),
            pltpu.SemaphoreType.REGULAR,
            pltpu.SemaphoreType.REGULAR,
            pltpu.SemaphoreType.DMA((N_HOPS,)),
            pltpu.SemaphoreType.DMA((N_HOPS,)),
            pltpu.SemaphoreType.DMA((N_HOPS,)),
            pltpu.SemaphoreType.DMA((N_HOPS,)),
            pltpu.SemaphoreType.DMA((N_HOPS,)),
            pltpu.SemaphoreType.DMA((N_HOPS,)),
            pltpu.SemaphoreType.DMA((N_HOPS,)),
            pltpu.SemaphoreType.DMA((N_HOPS,)),
        ],
        compiler_params=pltpu.CompilerParams(
            collective_id=0, vmem_limit_bytes=60 * 1024 * 1024
        ),
    )(Qt, KA, KB, VA, VB)
    return out


def kernel(Q, K, V):
    b, s, h, d = Q.shape
    BH = b * h
    H2 = s // 2

    def to_dmajor(x):
        return (
            x.transpose(0, 2, 3, 1)
            .reshape(BH, d, x.shape[1])
            .astype(jnp.bfloat16)
        )

    Qt = to_dmajor(Q)
    KA = to_dmajor(K[:, :H2])
    KB = to_dmajor(K[:, H2:])
    VA = to_dmajor(V[:, :H2])
    VB = to_dmajor(V[:, H2:])

    out = _fused_agattn(Qt, KA, KB, VA, VB, s)
    return out.reshape(b, h, s, d).transpose(0, 2, 1, 3).astype(Q.dtype)
